# Initial kernel scaffold; baseline (speedup 1.0000x reference)
import jax
import jax.numpy as jnp
from jax.experimental import pallas as pl


def kernel(preds, target, target_mask, mu, logvar, e_init, kl_weight):
    raise NotImplementedError("write your pallas kernel here")


# speedup vs baseline: 2.2177x; 2.2177x over previous
import functools

import jax
import jax.numpy as jnp
from jax.experimental import pallas as pl
from jax.experimental.pallas import tpu as pltpu

_LAMBDA_E_SUM = 10.0
_LAMBDA_HIT = 20.0
_LAMBDA_CHAMFER = 1e-3
_LAMBDA_HIT_ENTROPY = 0.1
_EPS = 1e-6
_BIG = 1e18

_TP = 256


def _stats_kernel(preds_ref, target_ref, mask_ref, mu_ref, logvar_ref,
                  e_ref, o_ref, *, np_pts, ng_pts):
    p5 = preds_ref[0]
    t4 = target_ref[0]
    m = mask_ref[0]

    pxyz = p5[0:3, :]
    p_e = p5[3:4, :]
    p_hit = p5[4:5, :]
    txyz = t4[0:3, :]
    t_e = t4[3:4, :]

    ones_p = jnp.ones((1, np_pts), jnp.float32)
    ones_t = jnp.ones((1, ng_pts), jnp.float32)
    pn = jnp.sum(pxyz * pxyz, axis=0, keepdims=True)
    tn = jnp.sum(txyz * txyz, axis=0, keepdims=True)

    a_aug = jnp.concatenate(
        [pn, ones_p, -2.0 * pxyz, jnp.zeros((3, np_pts), jnp.float32)], axis=0)
    b_aug = jnp.concatenate(
        [ones_t, tn, txyz, jnp.zeros((3, ng_pts), jnp.float32)], axis=0)

    mask0 = m == 0.0
    g_iota = jax.lax.broadcasted_iota(
        jnp.int32, (_TP, ng_pts), 1).astype(jnp.float32)
    ii = jax.lax.broadcasted_iota(jnp.int32, (_TP, _TP), 0)
    jj = jax.lax.broadcasted_iota(jnp.int32, (_TP, _TP), 1)
    ident = jnp.where(ii == jj, 1.0, 0.0)

    cm = jnp.full((1, ng_pts), jnp.inf, jnp.float32)
    s_mdp = jnp.zeros((1, 1), jnp.float32)
    s_abse = jnp.zeros((1, 1), jnp.float32)

    for t in range(np_pts // _TP):
        ts = slice(t * _TP, (t + 1) * _TP)
        d = jax.lax.dot_general(
            a_aug[:, ts], b_aug, (((0,), (0,)), ((), ())),
            preferred_element_type=jnp.float32)
        d = jnp.maximum(d, 0.0)
        cm = jnp.minimum(cm, jnp.min(d, axis=0, keepdims=True))
        md = jnp.where(mask0, _BIG, d)
        rm = jnp.min(md, axis=1, keepdims=True)
        s_mdp = s_mdp + jnp.sum(rm, axis=0, keepdims=True)
        idxv = jnp.where(md == rm, g_iota, 1e9)
        fi = jnp.min(idxv, axis=1, keepdims=True)
        sel_e = jnp.where(g_iota == fi, t_e, 0.0)
        m_e = jnp.sum(sel_e, axis=1, keepdims=True)
        p_e_col = jax.lax.dot_general(
            ident, p_e[:, ts], (((1,), (1,)), ((), ())),
            preferred_element_type=jnp.float32)
        s_abse = s_abse + jnp.sum(jnp.abs(p_e_col - m_e), axis=0,
                                  keepdims=True)

    s_mdt = jnp.sum(cm * m, axis=1, keepdims=True)
    s_mask = jnp.sum(m, axis=1, keepdims=True)
    s_hit = jnp.sum(p_hit, axis=1, keepdims=True)
    hd2 = (s_hit - s_mask) ** 2
    ent = jnp.sum(p_hit * jnp.log(p_hit + _EPS)
                  + (1.0 - p_hit) * jnp.log(1.0 - p_hit + _EPS),
                  axis=1, keepdims=True)
    ge = jnp.sum(p_e * p_hit, axis=1, keepdims=True)
    ge2 = (ge - e_ref[0]) ** 2
    mu = mu_ref[0]
    lv = logvar_ref[0]
    kld = jnp.sum(1.0 + lv - mu * mu - jnp.exp(lv), axis=1, keepdims=True)

    lane = jax.lax.broadcasted_iota(jnp.int32, (1, 128), 1)
    out = jnp.zeros((1, 128), jnp.float32)
    for k, v in enumerate([s_mdt, s_mask, s_mdp, s_abse, hd2, ent, ge2, kld]):
        out = jnp.where(lane == k, v, out)
    o_ref[0] = out


def _combine_kernel(s_ref, kw_ref, o_ref, *, b, np_pts):
    cs = jnp.sum(s_ref[...].reshape(b, 128), axis=0, keepdims=True)
    s_mdt = cs[:, 0:1]
    s_mask = cs[:, 1:2]
    s_mdp = cs[:, 2:3]
    s_abse = cs[:, 3:4]
    hd2 = cs[:, 4:5]
    ent = cs[:, 5:6]
    ge2 = cs[:, 6:7]
    kld = cs[:, 7:8]

    bn = float(b * np_pts)
    loss_chamfer = s_mdt / s_mask + s_mdp / bn
    loss_local_e = s_abse / bn
    loss_hit_count = hd2 / float(b)
    loss_hit_entropy = -ent / bn
    loss_global_e_sum = ge2 / float(b)
    loss_kld = -0.5 * kld / float(b)

    loss_chamf = loss_chamfer * _LAMBDA_CHAMFER
    losskld = kw_ref[...] * loss_kld
    loss_global_e = _LAMBDA_E_SUM * loss_global_e_sum
    loss_hit = _LAMBDA_HIT * loss_hit_count
    loss_hit_entr = _LAMBDA_HIT_ENTROPY * loss_hit_entropy
    total = (loss_chamf + loss_local_e + losskld + loss_global_e
             + loss_hit + loss_hit_entr)

    lane = jax.lax.broadcasted_iota(jnp.int32, (1, 128), 1)
    out = jnp.zeros((1, 128), jnp.float32)
    for k, v in enumerate([total, loss_chamf, loss_local_e, loss_global_e,
                           loss_hit, losskld]):
        out = jnp.where(lane == k, v, out)
    o_ref[...] = out


def kernel(preds, target, target_mask, mu, logvar, e_init, kl_weight):
    b, _, np_pts = preds.shape
    ng_pts = target.shape[2]
    l_dim = mu.shape[1]

    stats = pl.pallas_call(
        functools.partial(_stats_kernel, np_pts=np_pts, ng_pts=ng_pts),
        grid=(b,),
        in_specs=[
            pl.BlockSpec((1, 5, np_pts), lambda i: (i, 0, 0)),
            pl.BlockSpec((1, 4, ng_pts), lambda i: (i, 0, 0)),
            pl.BlockSpec((1, 1, ng_pts), lambda i: (i, 0, 0)),
            pl.BlockSpec((1, 1, l_dim), lambda i: (i, 0, 0)),
            pl.BlockSpec((1, 1, l_dim), lambda i: (i, 0, 0)),
            pl.BlockSpec((1, 1, 1), lambda i: (i, 0, 0)),
        ],
        out_specs=pl.BlockSpec((1, 1, 128), lambda i: (i, 0, 0)),
        out_shape=jax.ShapeDtypeStruct((b, 1, 128), jnp.float32),
        compiler_params=pltpu.CompilerParams(
            dimension_semantics=("parallel",)),
        name="pcvae_stats",
    )(preds, target, target_mask.reshape(b, 1, ng_pts),
      mu.reshape(b, 1, l_dim), logvar.reshape(b, 1, l_dim),
      e_init.reshape(b, 1, 1).astype(jnp.float32))

    out = pl.pallas_call(
        functools.partial(_combine_kernel, b=b, np_pts=np_pts),
        out_shape=jax.ShapeDtypeStruct((1, 128), jnp.float32),
        name="pcvae_combine",
    )(stats, jnp.asarray(kl_weight, jnp.float32).reshape(1, 1))

    return out[0, 0], out[0, 1:6]


# speedup vs baseline: 3.9284x; 1.7714x over previous
import functools

import jax
import jax.numpy as jnp
from jax.experimental import pallas as pl
from jax.experimental.pallas import tpu as pltpu

_LAMBDA_E_SUM = 10.0
_LAMBDA_HIT = 20.0
_LAMBDA_CHAMFER = 1e-3
_LAMBDA_HIT_ENTROPY = 0.1
_EPS = 1e-6
_BIG = 1e18

_TP = 256
_TC = 256
_NC = 8


def _stats_kernel(preds_ref, target_ref, mask_ref, mu_ref, logvar_ref,
                  e_ref, o_ref, *, np_pts, ng_pts):
    p5 = preds_ref[0]
    t4 = target_ref[0]
    m = mask_ref[0]

    pxyz = p5[0:3, :]
    p_e = p5[3:4, :]
    p_hit = p5[4:5, :]
    txyz = t4[0:3, :]
    t_e = t4[3:4, :]

    ones_p = jnp.ones((1, np_pts), jnp.float32)
    ones_t = jnp.ones((1, ng_pts), jnp.float32)
    pn = jnp.sum(pxyz * pxyz, axis=0, keepdims=True)
    tn = jnp.sum(txyz * txyz, axis=0, keepdims=True)

    pen = jnp.where(m == 0.0, _BIG, 0.0)
    a_aug = jnp.concatenate(
        [pn, ones_p, -2.0 * pxyz, jnp.zeros((3, np_pts), jnp.float32)], axis=0)
    b_aug = jnp.concatenate(
        [ones_t, tn + pen, txyz, jnp.zeros((3, ng_pts), jnp.float32)], axis=0)

    ii = jax.lax.broadcasted_iota(jnp.int32, (_TP, _TP), 0)
    jj = jax.lax.broadcasted_iota(jnp.int32, (_TP, _TP), 1)
    ident = jnp.where(ii == jj, 1.0, 0.0)

    cm = jnp.full((1, ng_pts), jnp.inf, jnp.float32)
    s_mdp = jnp.zeros((1, 1), jnp.float32)
    s_abse = jnp.zeros((1, 1), jnp.float32)

    for t in range(np_pts // _TP):
        ts = slice(t * _TP, (t + 1) * _TP)
        md = jax.lax.dot_general(
            a_aug[:, ts], b_aug, (((0,), (0,)), ((), ())),
            preferred_element_type=jnp.float32)
        cm = jnp.minimum(cm, jnp.min(md, axis=0, keepdims=True))
        rm = jnp.min(md, axis=1, keepdims=True)
        s_mdp = s_mdp + jnp.sum(jnp.maximum(rm, 0.0), axis=0, keepdims=True)
        m_e = jnp.sum(jnp.where(md == rm, t_e, 0.0),
                      axis=1, keepdims=True)
        p_e_col = jax.lax.dot_general(
            ident, p_e[:, ts], (((1,), (1,)), ((), ())),
            preferred_element_type=jnp.float32)
        s_abse = s_abse + jnp.sum(jnp.abs(p_e_col - m_e), axis=0,
                                  keepdims=True)

    s_mdt = jnp.sum(jnp.maximum(cm, 0.0) * m, axis=1, keepdims=True)
    s_mask = jnp.sum(m, axis=1, keepdims=True)
    s_hit = jnp.sum(p_hit, axis=1, keepdims=True)
    hd2 = (s_hit - s_mask) ** 2
    ent = jnp.sum(p_hit * jnp.log(p_hit + _EPS)
                  + (1.0 - p_hit) * jnp.log(1.0 - p_hit + _EPS),
                  axis=1, keepdims=True)
    ge = jnp.sum(p_e * p_hit, axis=1, keepdims=True)
    ge2 = (ge - e_ref[0]) ** 2
    mu = mu_ref[0]
    lv = logvar_ref[0]
    kld = jnp.sum(1.0 + lv - mu * mu - jnp.exp(lv), axis=1, keepdims=True)

    lane = jax.lax.broadcasted_iota(jnp.int32, (1, 128), 1)
    out = jnp.zeros((1, 128), jnp.float32)
    for k, v in enumerate([s_mdt, s_mask, s_mdp, s_abse, hd2, ent, ge2, kld]):
        out = jnp.where(lane == k, v, out)
    o_ref[0] = out


def _combine_kernel(s_ref, kw_ref, o_ref, *, b, np_pts):
    cs = jnp.sum(s_ref[...].reshape(b, 128), axis=0, keepdims=True)
    s_mdt = cs[:, 0:1]
    s_mask = cs[:, 1:2]
    s_mdp = cs[:, 2:3]
    s_abse = cs[:, 3:4]
    hd2 = cs[:, 4:5]
    ent = cs[:, 5:6]
    ge2 = cs[:, 6:7]
    kld = cs[:, 7:8]

    bn = float(b * np_pts)
    loss_chamfer = s_mdt / s_mask + s_mdp / bn
    loss_local_e = s_abse / bn
    loss_hit_count = hd2 / float(b)
    loss_hit_entropy = -ent / bn
    loss_global_e_sum = ge2 / float(b)
    loss_kld = -0.5 * kld / float(b)

    loss_chamf = loss_chamfer * _LAMBDA_CHAMFER
    losskld = kw_ref[...] * loss_kld
    loss_global_e = _LAMBDA_E_SUM * loss_global_e_sum
    loss_hit = _LAMBDA_HIT * loss_hit_count
    loss_hit_entr = _LAMBDA_HIT_ENTROPY * loss_hit_entropy
    total = (loss_chamf + loss_local_e + losskld + loss_global_e
             + loss_hit + loss_hit_entr)

    lane = jax.lax.broadcasted_iota(jnp.int32, (1, 128), 1)
    out = jnp.zeros((1, 128), jnp.float32)
    for k, v in enumerate([total, loss_chamf, loss_local_e, loss_global_e,
                           loss_hit, losskld]):
        out = jnp.where(lane == k, v, out)
    o_ref[...] = out


def kernel(preds, target, target_mask, mu, logvar, e_init, kl_weight):
    b, _, np_pts = preds.shape
    ng_pts = target.shape[2]
    l_dim = mu.shape[1]

    def bmap(i):
        return (i, 0, 0)

    stats = pl.pallas_call(
        functools.partial(_stats_kernel, np_pts=np_pts, ng_pts=ng_pts),
        grid=(b,),
        in_specs=[
            pl.BlockSpec((1, 5, np_pts), bmap),
            pl.BlockSpec((1, 4, ng_pts), bmap),
            pl.BlockSpec((1, 1, ng_pts), bmap),
            pl.BlockSpec((1, 1, l_dim), bmap),
            pl.BlockSpec((1, 1, l_dim), bmap),
            pl.BlockSpec((1, 1, 1), bmap),
        ],
        out_specs=pl.BlockSpec((1, 1, 128), bmap),
        out_shape=jax.ShapeDtypeStruct((b, 1, 128), jnp.float32),
        compiler_params=pltpu.CompilerParams(
            dimension_semantics=("parallel",)),
        name="pcvae_stats",
    )(preds, target, target_mask.reshape(b, 1, ng_pts),
      mu.reshape(b, 1, l_dim), logvar.reshape(b, 1, l_dim),
      e_init.reshape(b, 1, 1).astype(jnp.float32))

    out = pl.pallas_call(
        functools.partial(_combine_kernel, b=b, np_pts=np_pts),
        out_shape=jax.ShapeDtypeStruct((1, 128), jnp.float32),
        name="pcvae_combine",
    )(stats, jnp.asarray(kl_weight, jnp.float32).reshape(1, 1))

    return out[0, 0], out[0, 1:6]
